# Initial kernel scaffold; baseline (speedup 1.0000x reference)
import jax
import jax.numpy as jnp
from jax.experimental import pallas as pl


def kernel(edge_attr, edge_type, layer_embed_w, W_proj, b_proj, W1, b1, W2, b2, W3, b3, k):
    raise NotImplementedError("write your pallas kernel here")


# speedup vs baseline: 1.4995x; 1.4995x over previous
import jax
import jax.numpy as jnp
from jax.experimental import pallas as pl

E = 160000
CTX = 5
HID = 128
OUT_DIM = 40
BE = 2000


def _mlp_body(x_ref, et_ref, wc_ref, bc_ref, t2_ref, w2_ref, b2_ref,
              w3_ref, b3_ref, o_ref):
    x = x_ref[...]
    et = et_ref[...]
    onehot = (et == jax.lax.broadcasted_iota(jnp.int32, (1, 16), 1)
              ).astype(jnp.float32)
    z1 = jnp.dot(x, wc_ref[...], preferred_element_type=jnp.float32)
    z1 = z1 + jnp.dot(onehot, t2_ref[...], preferred_element_type=jnp.float32)
    z1 = z1 + bc_ref[...]
    h1 = z1 * jax.nn.sigmoid(z1)
    z2 = jnp.dot(h1, w2_ref[...], preferred_element_type=jnp.float32) + b2_ref[...]
    h2 = z2 * jax.nn.sigmoid(z2)
    o_ref[...] = jnp.dot(h2, w3_ref[...], preferred_element_type=jnp.float32) + b3_ref[...]


def kernel(edge_attr, edge_type, layer_embed_w, W_proj, b_proj,
           W1, b1, W2, b2, W3, b3, k=1):
    e = edge_attr.shape[0]
    wc = jnp.dot(W1, W_proj)
    wc_t = jnp.zeros((8, HID), jnp.float32).at[:CTX, :].set(wc.T)
    bc = (jnp.dot(W1, b_proj) + b1).reshape(1, HID)
    t2 = jnp.dot(layer_embed_w, W1.T)
    t2p = jnp.zeros((16, HID), jnp.float32).at[:15, :].set(t2)
    w2_t = W2.T
    b2r = b2.reshape(1, HID)
    w3_t = W3.T
    b3r = b3.reshape(1, OUT_DIM)
    x = jnp.pad(edge_attr, ((0, 0), (0, 8 - CTX)))
    et = edge_type.astype(jnp.int32).reshape(e, 1)

    grid = (e // BE,)
    rep = lambda i: (0, 0)
    out = pl.pallas_call(
        _mlp_body,
        grid=grid,
        in_specs=[
            pl.BlockSpec((BE, 8), lambda i: (i, 0)),
            pl.BlockSpec((BE, 1), lambda i: (i, 0)),
            pl.BlockSpec((8, HID), rep),
            pl.BlockSpec((1, HID), rep),
            pl.BlockSpec((16, HID), rep),
            pl.BlockSpec((HID, HID), rep),
            pl.BlockSpec((1, HID), rep),
            pl.BlockSpec((HID, OUT_DIM), rep),
            pl.BlockSpec((1, OUT_DIM), rep),
        ],
        out_specs=pl.BlockSpec((BE, OUT_DIM), lambda i: (i, 0)),
        out_shape=jax.ShapeDtypeStruct((e, OUT_DIM), jnp.float32),
    )(x, et, wc_t, bc, t2p, w2_t, b2r, w3_t, b3r)
    return out.reshape(e, 1, OUT_DIM)
